# Initial kernel scaffold; baseline (speedup 1.0000x reference)
import functools

import jax
import jax.numpy as jnp
from jax import lax
from jax.experimental import pallas as pl
from jax.experimental.pallas import tpu as pltpu
from jax.experimental.pallas import tpu_sc as plsc

N = 10000
E = 320000
NC = 2
NS = 16
NW = NC * NS
CH = 128
NCHUNK = -(-E // (NW * CH))
EPW = NCHUNK * CH
EPAD = EPW * NW
R = 10240
RPT = R // NS

_MESH = plsc.VectorSubcoreMesh(
    core_axis_name="c", subcore_axis_name="s", num_cores=NC, num_subcores=NS
)


def _make_scatter(D):
  @functools.partial(
      pl.kernel,
      out_type=jax.ShapeDtypeStruct((NC, R, D), jnp.float32),
      mesh=_MESH,
      scratch_types=[
          pltpu.VMEM((NCHUNK, CH), jnp.int32),
          pltpu.VMEM((NCHUNK, CH), jnp.int32),
          pltpu.VMEM((CH, D), jnp.float32),
          pltpu.VMEM_SHARED((R, D), jnp.float32),
          pltpu.SemaphoreType.DMA,
      ],
  )
  def scatter(g_hbm, src_hbm, dst_hbm, zero_hbm, out_hbm,
              src_v, dst_v, rows_v, acc_sh, sem):
    cid = lax.axis_index("c")
    sid = lax.axis_index("s")
    wid = cid * NS + sid
    pltpu.sync_copy(src_hbm.at[wid], src_v)
    pltpu.sync_copy(dst_hbm.at[wid], dst_v)
    pltpu.sync_copy(zero_hbm, acc_sh.at[pl.ds(sid * RPT, RPT)])
    plsc.subcore_barrier()

    def chunk(j, carry):
      pltpu.async_copy(g_hbm.at[src_v.at[j]], rows_v, sem).wait()
      pltpu.sync_copy(rows_v, acc_sh.at[dst_v.at[j]], add=True)
      return carry

    lax.fori_loop(0, NCHUNK, chunk, 0)
    plsc.subcore_barrier()
    pltpu.sync_copy(acc_sh.at[pl.ds(sid * RPT, RPT)],
                    out_hbm.at[cid, pl.ds(sid * RPT, RPT)])

  return scatter


_scatter16 = _make_scatter(16)
_scatter64 = _make_scatter(64)


def _k0_body(x_ref, dega_ref, degb_ref, w_ref, h_ref, g_ref, dis_ref, dinv_ref):
  deg = dega_ref[...] + degb_ref[...] + 1.0
  dis = lax.rsqrt(deg)
  dinv = 1.0 / deg
  h = jnp.dot(x_ref[...], w_ref[...], preferred_element_type=jnp.float32)
  h_ref[...] = h
  g_ref[...] = h * dis
  dis_ref[...] = dis
  dinv_ref[...] = dinv


def _mid_body(sa_ref, sb_ref, h_ref, dis_ref, dinv_ref, b_ref, w_ref,
              hn_ref, gn_ref):
  dis = dis_ref[...]
  a = dis * (sa_ref[...] + sb_ref[...]) + dinv_ref[...] * h_ref[...] + b_ref[...]
  a = jnp.maximum(a, 0.0)
  h = jnp.dot(a, w_ref[...], preferred_element_type=jnp.float32)
  hn_ref[...] = h
  gn_ref[...] = h * dis


def _k3_body(sa_ref, sb_ref, h_ref, dis_ref, dinv_ref, b_ref, out_ref):
  out_ref[...] = (dis_ref[...] * (sa_ref[...] + sb_ref[...])
                  + dinv_ref[...] * h_ref[...] + b_ref[...])


def _f32(shape):
  return jax.ShapeDtypeStruct(shape, jnp.float32)


_k0 = pl.pallas_call(
    _k0_body,
    out_shape=(_f32((R, 64)), _f32((R, 64)), _f32((R, 1)), _f32((R, 1))),
)


def _mid(dout):
  return pl.pallas_call(_mid_body, out_shape=(_f32((R, dout)), _f32((R, dout))))


_k1 = _mid(64)
_k2 = _mid(16)
_k3 = pl.pallas_call(_k3_body, out_shape=_f32((R, 16)))


def kernel(x, edge_index, W1, b1, W2, b2, W3, b3):
  src = edge_index[0].astype(jnp.int32)
  dst = edge_index[1].astype(jnp.int32)
  pad = jnp.full((EPAD - E,), N, dtype=jnp.int32)
  src = jnp.concatenate([src, pad]).reshape(NW, NCHUNK, CH)
  dst = jnp.concatenate([dst, pad]).reshape(NW, NCHUNK, CH)

  xp = jnp.zeros((R, 128), jnp.float32).at[:N].set(x)
  ones16 = jnp.ones((R, 16), jnp.float32)
  zero16 = jnp.zeros((RPT, 16), jnp.float32)
  zero64 = jnp.zeros((RPT, 64), jnp.float32)

  degp = _scatter16(ones16, src, dst, zero16)
  dega = degp[0, :, 0:1]
  degb = degp[1, :, 0:1]

  w3p = jnp.zeros((64, 16), jnp.float32).at[:, :6].set(W3)
  b1r = b1.reshape(1, 64)
  b2r = b2.reshape(1, 64)
  b3r = jnp.zeros((1, 16), jnp.float32).at[0, :6].set(b3)

  h1, g1, dis, dinv = _k0(xp, dega, degb, W1)
  s1 = _scatter64(g1, src, dst, zero64)
  h2, g2 = _k1(s1[0], s1[1], h1, dis, dinv, b1r, W2)
  s2 = _scatter64(g2, src, dst, zero64)
  h3, g3 = _k2(s2[0], s2[1], h2, dis, dinv, b2r, w3p)
  s3 = _scatter16(g3, src, dst, zero16)
  out = _k3(s3[0], s3[1], h3, dis, dinv, b3r)
  return out[:N, :6]


# speedup vs baseline: 15.4922x; 15.4922x over previous
import functools

import jax
import jax.numpy as jnp
from jax import lax
from jax.experimental import pallas as pl
from jax.experimental.pallas import tpu as pltpu
from jax.experimental.pallas import tpu_sc as plsc

N = 10000
E = 320000
NC = 2
NS = 16
NW = NC * NS
CH = 128
NCHUNK = -(-E // (NW * CH))
EPW = NCHUNK * CH
EPAD = EPW * NW
R = 10240
RPT = R // NS

_MESH = plsc.VectorSubcoreMesh(
    core_axis_name="c", subcore_axis_name="s", num_cores=NC, num_subcores=NS
)


def _make_scatter(D):
  @functools.partial(
      pl.kernel,
      out_type=jax.ShapeDtypeStruct((NC, R, D), jnp.float32),
      mesh=_MESH,
      compiler_params=pltpu.CompilerParams(use_tc_tiling_on_sc=False),
      scratch_types=[
          pltpu.VMEM((NCHUNK, CH), jnp.int32),
          pltpu.VMEM((NCHUNK, CH), jnp.int32),
          pltpu.VMEM((CH, D), jnp.float32),
          pltpu.VMEM_SHARED((R, D), jnp.float32),
          pltpu.SemaphoreType.DMA,
      ],
  )
  def scatter(g_hbm, src_hbm, dst_hbm, zero_hbm, out_hbm,
              src_v, dst_v, rows_v, acc_sh, sem):
    cid = lax.axis_index("c")
    sid = lax.axis_index("s")
    wid = cid * NS + sid
    pltpu.sync_copy(src_hbm.at[wid], src_v)
    pltpu.sync_copy(dst_hbm.at[wid], dst_v)
    pltpu.sync_copy(zero_hbm, acc_sh.at[pl.ds(sid * RPT, RPT)])
    plsc.subcore_barrier()

    def chunk(j, carry):
      pltpu.async_copy(g_hbm.at[src_v.at[j]], rows_v, sem).wait()
      pltpu.sync_copy(rows_v, acc_sh.at[dst_v.at[j]], add=True)
      return carry

    lax.fori_loop(0, NCHUNK, chunk, 0)
    plsc.subcore_barrier()
    pltpu.sync_copy(acc_sh.at[pl.ds(sid * RPT, RPT)],
                    out_hbm.at[cid, pl.ds(sid * RPT, RPT)])

  return scatter


_scatter16 = _make_scatter(16)
_scatter64 = _make_scatter(64)


def _k0_body(x_ref, dega_ref, degb_ref, w_ref, h_ref, g_ref, dis_ref, dinv_ref):
  deg = dega_ref[...] + degb_ref[...] + 1.0
  dis = lax.rsqrt(deg)
  dinv = 1.0 / deg
  h = jnp.dot(x_ref[...], w_ref[...], preferred_element_type=jnp.float32)
  h_ref[...] = h
  g_ref[...] = h * dis
  dis_ref[...] = dis
  dinv_ref[...] = dinv


def _mid_body(sa_ref, sb_ref, h_ref, dis_ref, dinv_ref, b_ref, w_ref,
              hn_ref, gn_ref):
  dis = dis_ref[...]
  a = dis * (sa_ref[...] + sb_ref[...]) + dinv_ref[...] * h_ref[...] + b_ref[...]
  a = jnp.maximum(a, 0.0)
  h = jnp.dot(a, w_ref[...], preferred_element_type=jnp.float32)
  hn_ref[...] = h
  gn_ref[...] = h * dis


def _k3_body(sa_ref, sb_ref, h_ref, dis_ref, dinv_ref, b_ref, out_ref):
  out_ref[...] = (dis_ref[...] * (sa_ref[...] + sb_ref[...])
                  + dinv_ref[...] * h_ref[...] + b_ref[...])


def _f32(shape):
  return jax.ShapeDtypeStruct(shape, jnp.float32)


_k0 = pl.pallas_call(
    _k0_body,
    out_shape=(_f32((R, 64)), _f32((R, 64)), _f32((R, 1)), _f32((R, 1))),
)


def _mid(dout):
  return pl.pallas_call(_mid_body, out_shape=(_f32((R, dout)), _f32((R, dout))))


_k1 = _mid(64)
_k2 = _mid(16)
_k3 = pl.pallas_call(_k3_body, out_shape=_f32((R, 16)))


def kernel(x, edge_index, W1, b1, W2, b2, W3, b3):
  src = edge_index[0].astype(jnp.int32)
  dst = edge_index[1].astype(jnp.int32)
  pad = jnp.full((EPAD - E,), N, dtype=jnp.int32)
  src = jnp.concatenate([src, pad]).reshape(NW, NCHUNK, CH)
  dst = jnp.concatenate([dst, pad]).reshape(NW, NCHUNK, CH)

  xp = jnp.zeros((R, 128), jnp.float32).at[:N].set(x)
  ones16 = jnp.ones((R, 16), jnp.float32)
  zero16 = jnp.zeros((RPT, 16), jnp.float32)
  zero64 = jnp.zeros((RPT, 64), jnp.float32)

  degp = _scatter16(ones16, src, dst, zero16)
  dega = degp[0, :, 0:1]
  degb = degp[1, :, 0:1]

  w3p = jnp.zeros((64, 16), jnp.float32).at[:, :6].set(W3)
  b1r = b1.reshape(1, 64)
  b2r = b2.reshape(1, 64)
  b3r = jnp.zeros((1, 16), jnp.float32).at[0, :6].set(b3)

  h1, g1, dis, dinv = _k0(xp, dega, degb, W1)
  s1 = _scatter64(g1, src, dst, zero64)
  h2, g2 = _k1(s1[0], s1[1], h1, dis, dinv, b1r, W2)
  s2 = _scatter64(g2, src, dst, zero64)
  h3, g3 = _k2(s2[0], s2[1], h2, dis, dinv, b2r, w3p)
  s3 = _scatter16(g3, src, dst, zero16)
  out = _k3(s3[0], s3[1], h3, dis, dinv, b3r)
  return out[:N, :6]


# speedup vs baseline: 16.0449x; 1.0357x over previous
import functools

import jax
import jax.numpy as jnp
from jax import lax
from jax.experimental import pallas as pl
from jax.experimental.pallas import tpu as pltpu
from jax.experimental.pallas import tpu_sc as plsc

N = 10000
E = 320000
NC = 2
NS = 16
NW = NC * NS
CH = 128
NBUF = 4
NCHUNK = 80
EPW = NCHUNK * CH
EPAD = EPW * NW
NQ = NCHUNK // NBUF
R = 10240
RPT = R // NS

_MESH = plsc.VectorSubcoreMesh(
    core_axis_name="c", subcore_axis_name="s", num_cores=NC, num_subcores=NS
)


def _make_scatter(D, ones_mode=False):
  row_bufs = [pltpu.VMEM((CH, D), jnp.float32) for _ in range(NBUF)]
  gsems = [pltpu.SemaphoreType.DMA for _ in range(NBUF)]
  ssems = [pltpu.SemaphoreType.DMA for _ in range(NBUF)]

  @functools.partial(
      pl.kernel,
      out_type=jax.ShapeDtypeStruct((NC, R, D), jnp.float32),
      mesh=_MESH,
      compiler_params=pltpu.CompilerParams(use_tc_tiling_on_sc=False),
      scratch_types=[
          pltpu.VMEM((NCHUNK, CH), jnp.int32),
          pltpu.VMEM((NCHUNK, CH), jnp.int32),
          pltpu.VMEM_SHARED((R, D), jnp.float32),
      ] + row_bufs + gsems + ssems,
  )
  def scatter(g_hbm, src_hbm, dst_hbm, zero_hbm, out_hbm,
              src_v, dst_v, acc_sh, *bufs_and_sems):
    rows = bufs_and_sems[:NBUF]
    gsem = bufs_and_sems[NBUF:2 * NBUF]
    ssem = bufs_and_sems[2 * NBUF:]
    cid = lax.axis_index("c")
    sid = lax.axis_index("s")
    wid = cid * NS + sid
    pltpu.sync_copy(src_hbm.at[wid], src_v)
    pltpu.sync_copy(dst_hbm.at[wid], dst_v)
    pltpu.sync_copy(zero_hbm, acc_sh.at[pl.ds(sid * RPT, RPT)])

    if ones_mode:
      for r in range(CH):
        rows[0][r, :] = jnp.ones((D,), jnp.float32)
    plsc.subcore_barrier()

    if ones_mode:
      def quad(q, carry):
        for b in range(NBUF):
          j = NBUF * q + b
          pltpu.async_copy(rows[0], acc_sh.at[dst_v.at[j]], ssem[b], add=True)
        for b in range(NBUF):
          j = NBUF * q + b
          pltpu.make_async_copy(rows[0], acc_sh.at[dst_v.at[j]],
                                ssem[b]).wait()
        return carry

      lax.fori_loop(0, NQ, quad, 0)
    else:
      for b in range(NBUF):
        pltpu.async_copy(g_hbm.at[src_v.at[b]], rows[b], gsem[b])

      def quad(q, carry):
        for b in range(NBUF):
          j = NBUF * q + b
          pltpu.make_async_copy(g_hbm.at[src_v.at[j]], rows[b],
                                gsem[b]).wait()
          pltpu.async_copy(rows[b], acc_sh.at[dst_v.at[j]], ssem[b], add=True)
        for b in range(NBUF):
          j = NBUF * q + b
          pltpu.make_async_copy(rows[b], acc_sh.at[dst_v.at[j]],
                                ssem[b]).wait()
          pltpu.async_copy(g_hbm.at[src_v.at[j + NBUF]], rows[b], gsem[b])
        return carry

      lax.fori_loop(0, NQ - 1, quad, 0)
      for b in range(NBUF):
        j = NBUF * (NQ - 1) + b
        pltpu.make_async_copy(g_hbm.at[src_v.at[j]], rows[b], gsem[b]).wait()
        pltpu.async_copy(rows[b], acc_sh.at[dst_v.at[j]], ssem[b], add=True)
      for b in range(NBUF):
        j = NBUF * (NQ - 1) + b
        pltpu.make_async_copy(rows[b], acc_sh.at[dst_v.at[j]], ssem[b]).wait()

    plsc.subcore_barrier()
    pltpu.sync_copy(acc_sh.at[pl.ds(sid * RPT, RPT)],
                    out_hbm.at[cid, pl.ds(sid * RPT, RPT)])

  return scatter


_scatter_deg = _make_scatter(16, ones_mode=True)
_scatter16 = _make_scatter(16)
_scatter64 = _make_scatter(64)


def _k0_body(x_ref, dega_ref, degb_ref, w_ref, h_ref, g_ref, dis_ref, dinv_ref):
  deg = dega_ref[...] + degb_ref[...] + 1.0
  dis = lax.rsqrt(deg)
  dinv = 1.0 / deg
  h = jnp.dot(x_ref[...], w_ref[...], preferred_element_type=jnp.float32)
  h_ref[...] = h
  g_ref[...] = h * dis
  dis_ref[...] = dis
  dinv_ref[...] = dinv


def _mid_body(sa_ref, sb_ref, h_ref, dis_ref, dinv_ref, b_ref, w_ref,
              hn_ref, gn_ref):
  dis = dis_ref[...]
  a = dis * (sa_ref[...] + sb_ref[...]) + dinv_ref[...] * h_ref[...] + b_ref[...]
  a = jnp.maximum(a, 0.0)
  h = jnp.dot(a, w_ref[...], preferred_element_type=jnp.float32)
  hn_ref[...] = h
  gn_ref[...] = h * dis


def _k3_body(sa_ref, sb_ref, h_ref, dis_ref, dinv_ref, b_ref, out_ref):
  out_ref[...] = (dis_ref[...] * (sa_ref[...] + sb_ref[...])
                  + dinv_ref[...] * h_ref[...] + b_ref[...])


def _f32(shape):
  return jax.ShapeDtypeStruct(shape, jnp.float32)


_k0 = pl.pallas_call(
    _k0_body,
    out_shape=(_f32((R, 64)), _f32((R, 64)), _f32((R, 1)), _f32((R, 1))),
)


def _mid(dout):
  return pl.pallas_call(_mid_body, out_shape=(_f32((R, dout)), _f32((R, dout))))


_k1 = _mid(64)
_k2 = _mid(16)
_k3 = pl.pallas_call(_k3_body, out_shape=_f32((R, 16)))


def kernel(x, edge_index, W1, b1, W2, b2, W3, b3):
  src = edge_index[0].astype(jnp.int32)
  dst = edge_index[1].astype(jnp.int32)
  pad = jnp.full((EPAD - E,), N, dtype=jnp.int32)
  src = jnp.concatenate([src, pad]).reshape(NW, NCHUNK, CH)
  dst = jnp.concatenate([dst, pad]).reshape(NW, NCHUNK, CH)

  xp = jnp.zeros((R, 128), jnp.float32).at[:N].set(x)
  ones16 = jnp.ones((R, 16), jnp.float32)
  zero16 = jnp.zeros((RPT, 16), jnp.float32)
  zero64 = jnp.zeros((RPT, 64), jnp.float32)

  degp = _scatter_deg(ones16, src, dst, zero16)
  dega = degp[0, :, 0:1]
  degb = degp[1, :, 0:1]

  w3p = jnp.zeros((64, 16), jnp.float32).at[:, :6].set(W3)
  b1r = b1.reshape(1, 64)
  b2r = b2.reshape(1, 64)
  b3r = jnp.zeros((1, 16), jnp.float32).at[0, :6].set(b3)

  h1, g1, dis, dinv = _k0(xp, dega, degb, W1)
  s1 = _scatter64(g1, src, dst, zero64)
  h2, g2 = _k1(s1[0], s1[1], h1, dis, dinv, b1r, W2)
  s2 = _scatter64(g2, src, dst, zero64)
  h3, g3 = _k2(s2[0], s2[1], h2, dis, dinv, b2r, w3p)
  s3 = _scatter16(g3, src, dst, zero16)
  out = _k3(s3[0], s3[1], h3, dis, dinv, b3r)
  return out[:N, :6]


# speedup vs baseline: 27.2324x; 1.6973x over previous
import functools

import jax
import jax.numpy as jnp
from jax import lax
from jax.experimental import pallas as pl
from jax.experimental.pallas import tpu as pltpu
from jax.experimental.pallas import tpu_sc as plsc

N = 10000
E = 320000
NC = 2
NS = 16
NW = NC * NS
CH = 128
NBUF = 2
NCHUNK = 80
EPW = NCHUNK * CH
EPAD = EPW * NW
NQ = NCHUNK // NBUF
R = 10240
RPT = R // NS

_MESH = plsc.VectorSubcoreMesh(
    core_axis_name="c", subcore_axis_name="s", num_cores=NC, num_subcores=NS
)


def _make_scatter(D, ones_mode=False, stage_table=False):
  row_bufs = [pltpu.VMEM((CH, D), jnp.float32) for _ in range(NBUF)]
  gsems = [pltpu.SemaphoreType.DMA for _ in range(NBUF)]
  ssems = [pltpu.SemaphoreType.DMA for _ in range(NBUF)]

  @functools.partial(
      pl.kernel,
      out_type=jax.ShapeDtypeStruct((NC, R, D), jnp.float32),
      mesh=_MESH,
      compiler_params=pltpu.CompilerParams(use_tc_tiling_on_sc=False),
      scratch_types=[
          pltpu.VMEM((NCHUNK, CH), jnp.int32),
          pltpu.VMEM((NCHUNK, CH), jnp.int32),
          pltpu.VMEM_SHARED((R, D), jnp.float32),
      ] + ([pltpu.VMEM_SHARED((R, D), jnp.float32)] if stage_table else [])
        + row_bufs + gsems + ssems,
  )
  def scatter(g_hbm, src_hbm, dst_hbm, zero_hbm, out_hbm,
              src_v, dst_v, acc_sh, *rest):
    if stage_table:
      g_sh, rest = rest[0], rest[1:]
    else:
      g_sh = g_hbm
    rows = rest[:NBUF]
    gsem = rest[NBUF:2 * NBUF]
    ssem = rest[2 * NBUF:]
    cid = lax.axis_index("c")
    sid = lax.axis_index("s")
    wid = cid * NS + sid
    pltpu.sync_copy(src_hbm.at[wid], src_v)
    pltpu.sync_copy(dst_hbm.at[wid], dst_v)
    pltpu.sync_copy(zero_hbm, acc_sh.at[pl.ds(sid * RPT, RPT)])
    if stage_table:
      pltpu.sync_copy(g_hbm.at[pl.ds(sid * RPT, RPT)],
                      g_sh.at[pl.ds(sid * RPT, RPT)])

    if ones_mode:
      for r in range(CH):
        rows[0][r, :] = jnp.ones((D,), jnp.float32)
    plsc.subcore_barrier()

    if ones_mode:
      def quad(q, carry):
        for b in range(NBUF):
          j = NBUF * q + b
          pltpu.async_copy(rows[0], acc_sh.at[dst_v.at[j]], ssem[b], add=True)
        for b in range(NBUF):
          j = NBUF * q + b
          pltpu.make_async_copy(rows[0], acc_sh.at[dst_v.at[j]],
                                ssem[b]).wait()
        return carry

      lax.fori_loop(0, NQ, quad, 0)
    else:
      for b in range(NBUF):
        pltpu.async_copy(g_sh.at[src_v.at[b]], rows[b], gsem[b])

      def quad(q, carry):
        for b in range(NBUF):
          j = NBUF * q + b
          pltpu.make_async_copy(g_sh.at[src_v.at[j]], rows[b],
                                gsem[b]).wait()
          pltpu.async_copy(rows[b], acc_sh.at[dst_v.at[j]], ssem[b], add=True)
        for b in range(NBUF):
          j = NBUF * q + b
          pltpu.make_async_copy(rows[b], acc_sh.at[dst_v.at[j]],
                                ssem[b]).wait()
          pltpu.async_copy(g_sh.at[src_v.at[j + NBUF]], rows[b], gsem[b])
        return carry

      lax.fori_loop(0, NQ - 1, quad, 0)
      for b in range(NBUF):
        j = NBUF * (NQ - 1) + b
        pltpu.make_async_copy(g_sh.at[src_v.at[j]], rows[b], gsem[b]).wait()
        pltpu.async_copy(rows[b], acc_sh.at[dst_v.at[j]], ssem[b], add=True)
      for b in range(NBUF):
        j = NBUF * (NQ - 1) + b
        pltpu.make_async_copy(rows[b], acc_sh.at[dst_v.at[j]], ssem[b]).wait()

    plsc.subcore_barrier()
    pltpu.sync_copy(acc_sh.at[pl.ds(sid * RPT, RPT)],
                    out_hbm.at[cid, pl.ds(sid * RPT, RPT)])

  return scatter


_scatter_deg = _make_scatter(16, ones_mode=True)
_scatter16 = _make_scatter(16)
_scatter64 = _make_scatter(64, stage_table=True)


def _k0_body(x_ref, dega_ref, degb_ref, w_ref, h_ref, g_ref, dis_ref, dinv_ref):
  deg = dega_ref[...] + degb_ref[...] + 1.0
  dis = lax.rsqrt(deg)
  dinv = 1.0 / deg
  h = jnp.dot(x_ref[...], w_ref[...], preferred_element_type=jnp.float32)
  h_ref[...] = h
  g_ref[...] = h * dis
  dis_ref[...] = dis
  dinv_ref[...] = dinv


def _mid_body(sa_ref, sb_ref, h_ref, dis_ref, dinv_ref, b_ref, w_ref,
              hn_ref, gn_ref):
  dis = dis_ref[...]
  a = dis * (sa_ref[...] + sb_ref[...]) + dinv_ref[...] * h_ref[...] + b_ref[...]
  a = jnp.maximum(a, 0.0)
  h = jnp.dot(a, w_ref[...], preferred_element_type=jnp.float32)
  hn_ref[...] = h
  gn_ref[...] = h * dis


def _k3_body(sa_ref, sb_ref, h_ref, dis_ref, dinv_ref, b_ref, out_ref):
  out_ref[...] = (dis_ref[...] * (sa_ref[...] + sb_ref[...])
                  + dinv_ref[...] * h_ref[...] + b_ref[...])


def _f32(shape):
  return jax.ShapeDtypeStruct(shape, jnp.float32)


_k0 = pl.pallas_call(
    _k0_body,
    out_shape=(_f32((R, 64)), _f32((R, 64)), _f32((R, 1)), _f32((R, 1))),
)


def _mid(dout):
  return pl.pallas_call(_mid_body, out_shape=(_f32((R, dout)), _f32((R, dout))))


_k1 = _mid(64)
_k2 = _mid(16)
_k3 = pl.pallas_call(_k3_body, out_shape=_f32((R, 16)))


def kernel(x, edge_index, W1, b1, W2, b2, W3, b3):
  src = edge_index[0].astype(jnp.int32)
  dst = edge_index[1].astype(jnp.int32)
  pad = jnp.full((EPAD - E,), N, dtype=jnp.int32)
  src = jnp.concatenate([src, pad]).reshape(NW, NCHUNK, CH)
  dst = jnp.concatenate([dst, pad]).reshape(NW, NCHUNK, CH)

  xp = jnp.zeros((R, 128), jnp.float32).at[:N].set(x)
  ones16 = jnp.ones((R, 16), jnp.float32)
  zero16 = jnp.zeros((RPT, 16), jnp.float32)
  zero64 = jnp.zeros((RPT, 64), jnp.float32)

  degp = _scatter_deg(ones16, src, dst, zero16)
  dega = degp[0, :, 0:1]
  degb = degp[1, :, 0:1]

  w3p = jnp.zeros((64, 16), jnp.float32).at[:, :6].set(W3)
  b1r = b1.reshape(1, 64)
  b2r = b2.reshape(1, 64)
  b3r = jnp.zeros((1, 16), jnp.float32).at[0, :6].set(b3)

  h1, g1, dis, dinv = _k0(xp, dega, degb, W1)
  s1 = _scatter64(g1, src, dst, zero64)
  h2, g2 = _k1(s1[0], s1[1], h1, dis, dinv, b1r, W2)
  s2 = _scatter64(g2, src, dst, zero64)
  h3, g3 = _k2(s2[0], s2[1], h2, dis, dinv, b2r, w3p)
  s3 = _scatter16(g3, src, dst, zero16)
  out = _k3(s3[0], s3[1], h3, dis, dinv, b3r)
  return out[:N, :6]


# speedup vs baseline: 29.7238x; 1.0915x over previous
import functools

import jax
import jax.numpy as jnp
from jax import lax
from jax.experimental import pallas as pl
from jax.experimental.pallas import tpu as pltpu
from jax.experimental.pallas import tpu_sc as plsc

N = 10000
E = 320000
NC = 2
NS = 16
NW = NC * NS
CH = 128
NBUF = 3
NCHUNK = 81
EPW = NCHUNK * CH
EPAD = EPW * NW
NQ = NCHUNK // NBUF
R = 10240
RPT = R // NS

_MESH = plsc.VectorSubcoreMesh(
    core_axis_name="c", subcore_axis_name="s", num_cores=NC, num_subcores=NS
)


def _make_scatter(D, ones_mode=False):
  row_bufs = [pltpu.VMEM((CH, D), jnp.float32) for _ in range(NBUF)]
  gsems = [pltpu.SemaphoreType.DMA for _ in range(NBUF)]
  ssems = [pltpu.SemaphoreType.DMA for _ in range(NBUF)]
  stage = [] if ones_mode else [
      pltpu.VMEM((NCHUNK, CH), jnp.int32),
      pltpu.VMEM_SHARED((R, D), jnp.float32),
  ]

  @functools.partial(
      pl.kernel,
      out_type=jax.ShapeDtypeStruct((NC, R, D), jnp.float32),
      mesh=_MESH,
      compiler_params=pltpu.CompilerParams(use_tc_tiling_on_sc=False),
      scratch_types=[
          pltpu.VMEM((NCHUNK, CH), jnp.int32),
          pltpu.VMEM_SHARED((R, D), jnp.float32),
      ] + stage + row_bufs + gsems + ssems,
  )
  def scatter(g_hbm, ei_hbm, zero_hbm, out_hbm, dst_v, acc_sh, *rest):
    if ones_mode:
      src_v = g_sh = None
    else:
      src_v, g_sh, rest = rest[0], rest[1], rest[2:]
    rows = rest[:NBUF]
    gsem = rest[NBUF:2 * NBUF]
    ssem = rest[2 * NBUF:]
    cid = lax.axis_index("c")
    sid = lax.axis_index("s")
    wid = cid * NS + sid
    pltpu.sync_copy(ei_hbm.at[1, wid], dst_v)
    if not ones_mode:
      pltpu.sync_copy(ei_hbm.at[0, wid], src_v)
      pltpu.sync_copy(g_hbm.at[pl.ds(sid * RPT, RPT)],
                      g_sh.at[pl.ds(sid * RPT, RPT)])
    pltpu.sync_copy(zero_hbm, acc_sh.at[pl.ds(sid * RPT, RPT)])

    if ones_mode:
      for r in range(CH):
        rows[0][r, :] = jnp.ones((D,), jnp.float32)
    plsc.subcore_barrier()

    if ones_mode:
      def step(q, carry):
        for b in range(NBUF):
          j = NBUF * q + b
          pltpu.async_copy(rows[0], acc_sh.at[dst_v.at[j]], ssem[b], add=True)
        for b in range(NBUF):
          j = NBUF * q + b
          pltpu.make_async_copy(rows[0], acc_sh.at[dst_v.at[j]],
                                ssem[b]).wait()
        return carry

      lax.fori_loop(0, NQ, step, 0)
    else:
      for b in range(NBUF):
        pltpu.async_copy(g_sh.at[src_v.at[b]], rows[b], gsem[b])

      def step(q, carry):
        for b in range(NBUF):
          j = NBUF * q + b
          pltpu.make_async_copy(g_sh.at[src_v.at[j]], rows[b],
                                gsem[b]).wait()
          pltpu.async_copy(rows[b], acc_sh.at[dst_v.at[j]], ssem[b], add=True)
        for b in range(NBUF):
          j = NBUF * q + b
          pltpu.make_async_copy(rows[b], acc_sh.at[dst_v.at[j]],
                                ssem[b]).wait()
          pltpu.async_copy(g_sh.at[src_v.at[j + NBUF]], rows[b], gsem[b])
        return carry

      lax.fori_loop(0, NQ - 1, step, 0)
      for b in range(NBUF):
        j = NBUF * (NQ - 1) + b
        pltpu.make_async_copy(g_sh.at[src_v.at[j]], rows[b], gsem[b]).wait()
        pltpu.async_copy(rows[b], acc_sh.at[dst_v.at[j]], ssem[b], add=True)
      for b in range(NBUF):
        j = NBUF * (NQ - 1) + b
        pltpu.make_async_copy(rows[b], acc_sh.at[dst_v.at[j]], ssem[b]).wait()

    plsc.subcore_barrier()
    pltpu.sync_copy(acc_sh.at[pl.ds(sid * RPT, RPT)],
                    out_hbm.at[cid, pl.ds(sid * RPT, RPT)])

  return scatter


_scatter_deg = _make_scatter(16, ones_mode=True)
_scatter16 = _make_scatter(16)
_scatter64 = _make_scatter(64)


def _k0_body(x_ref, degp_ref, w_ref, h_ref, g_ref, dis_ref, dinv_ref):
  deg = degp_ref[0, :, 0:1] + degp_ref[1, :, 0:1] + 1.0
  dis = lax.rsqrt(deg)
  dinv = 1.0 / deg
  h = jnp.dot(x_ref[...], w_ref[...], preferred_element_type=jnp.float32)
  h_ref[...] = h
  g_ref[...] = h * dis
  dis_ref[...] = dis
  dinv_ref[...] = dinv


def _mid_body(s_ref, h_ref, dis_ref, dinv_ref, b_ref, w_ref, hn_ref, gn_ref):
  dis = dis_ref[...]
  a = dis * (s_ref[0] + s_ref[1]) + dinv_ref[...] * h_ref[...] + b_ref[...]
  a = jnp.maximum(a, 0.0)
  h = jnp.dot(a, w_ref[...], preferred_element_type=jnp.float32)
  hn_ref[...] = h
  gn_ref[...] = h * dis


def _k3_body(s_ref, h_ref, dis_ref, dinv_ref, b_ref, out_ref):
  out_ref[...] = (dis_ref[...] * (s_ref[0] + s_ref[1])
                  + dinv_ref[...] * h_ref[...] + b_ref[...])


def _f32(shape):
  return jax.ShapeDtypeStruct(shape, jnp.float32)


_k0 = pl.pallas_call(
    _k0_body,
    out_shape=(_f32((R, 64)), _f32((R, 64)), _f32((R, 1)), _f32((R, 1))),
)


def _mid(dout):
  return pl.pallas_call(_mid_body, out_shape=(_f32((R, dout)), _f32((R, dout))))


_k1 = _mid(64)
_k2 = _mid(16)
_k3 = pl.pallas_call(_k3_body, out_shape=_f32((R, 16)))


def kernel(x, edge_index, W1, b1, W2, b2, W3, b3):
  ei = edge_index.astype(jnp.int32)
  ei = jnp.pad(ei, ((0, 0), (0, EPAD - E)), constant_values=N)
  ei = ei.reshape(2, NW, NCHUNK, CH)

  xp = jnp.zeros((R, 128), jnp.float32).at[:N].set(x)
  ones16 = jnp.ones((R, 16), jnp.float32)
  zero16 = jnp.zeros((RPT, 16), jnp.float32)
  zero64 = jnp.zeros((RPT, 64), jnp.float32)

  degp = _scatter_deg(ones16, ei, zero16)

  w3p = jnp.zeros((64, 16), jnp.float32).at[:, :6].set(W3)
  b1r = b1.reshape(1, 64)
  b2r = b2.reshape(1, 64)
  b3r = jnp.zeros((1, 16), jnp.float32).at[0, :6].set(b3)

  h1, g1, dis, dinv = _k0(xp, degp, W1)
  s1 = _scatter64(g1, ei, zero64)
  h2, g2 = _k1(s1, h1, dis, dinv, b1r, W2)
  s2 = _scatter64(g2, ei, zero64)
  h3, g3 = _k2(s2, h2, dis, dinv, b2r, w3p)
  s3 = _scatter16(g3, ei, zero16)
  out = _k3(s3, h3, dis, dinv, b3r)
  return out[:N, :6]


# speedup vs baseline: 33.1428x; 1.1150x over previous
import functools

import jax
import jax.numpy as jnp
from jax import lax
from jax.experimental import pallas as pl
from jax.experimental.pallas import tpu as pltpu
from jax.experimental.pallas import tpu_sc as plsc

N = 10000
E = 320000
NC = 2
NS = 16
NW = NC * NS
CH = 128
NBUF = 2
NCHUNK = 80
EPW = NCHUNK * CH
EPAD = EPW * NW
NQ = NCHUNK // NBUF
R = 10240
RPT = R // NS

_MESH = plsc.VectorSubcoreMesh(
    core_axis_name="c", subcore_axis_name="s", num_cores=NC, num_subcores=NS
)


def _make_scatter(D, ones_mode=False):
  row_bufs = [pltpu.VMEM((CH, D), jnp.float32) for _ in range(NBUF)]
  gsems = [pltpu.SemaphoreType.DMA for _ in range(NBUF)]
  ssems = [pltpu.SemaphoreType.DMA for _ in range(NBUF)]
  stage = [] if ones_mode else [
      pltpu.VMEM((NCHUNK, CH), jnp.int32),
      pltpu.VMEM_SHARED((R, D), jnp.float32),
  ]

  @functools.partial(
      pl.kernel,
      out_type=jax.ShapeDtypeStruct((NC, R, D), jnp.float32),
      mesh=_MESH,
      compiler_params=pltpu.CompilerParams(use_tc_tiling_on_sc=False),
      scratch_types=[
          pltpu.VMEM((NCHUNK, CH), jnp.int32),
          pltpu.VMEM_SHARED((R, D), jnp.float32),
      ] + stage + row_bufs + gsems + ssems,
  )
  def scatter(g_hbm, ei_hbm, zero_hbm, out_hbm, dst_v, acc_sh, *rest):
    if ones_mode:
      src_v = g_sh = None
    else:
      src_v, g_sh, rest = rest[0], rest[1], rest[2:]
    rows = rest[:NBUF]
    gsem = rest[NBUF:2 * NBUF]
    ssem = rest[2 * NBUF:]
    cid = lax.axis_index("c")
    sid = lax.axis_index("s")
    wid = cid * NS + sid
    pltpu.sync_copy(ei_hbm.at[1, wid], dst_v)
    if not ones_mode:
      pltpu.sync_copy(ei_hbm.at[0, wid], src_v)
      pltpu.sync_copy(g_hbm.at[pl.ds(sid * RPT, RPT)],
                      g_sh.at[pl.ds(sid * RPT, RPT)])
    pltpu.sync_copy(zero_hbm, acc_sh.at[pl.ds(sid * RPT, RPT)])

    if ones_mode:
      for r in range(CH):
        rows[0][r, :] = jnp.ones((D,), jnp.float32)
    plsc.subcore_barrier()

    if ones_mode:
      def step(q, carry):
        for b in range(NBUF):
          j = NBUF * q + b
          pltpu.async_copy(rows[0], acc_sh.at[dst_v.at[j]], ssem[b], add=True)
        for b in range(NBUF):
          j = NBUF * q + b
          pltpu.make_async_copy(rows[0], acc_sh.at[dst_v.at[j]],
                                ssem[b]).wait()
        return carry

      lax.fori_loop(0, NQ, step, 0)
    else:
      for b in range(NBUF):
        pltpu.async_copy(g_sh.at[src_v.at[b]], rows[b], gsem[b])

      def step(q, carry):
        for b in range(NBUF):
          j = NBUF * q + b
          pltpu.make_async_copy(g_sh.at[src_v.at[j]], rows[b],
                                gsem[b]).wait()
          pltpu.async_copy(rows[b], acc_sh.at[dst_v.at[j]], ssem[b], add=True)
        for b in range(NBUF):
          j = NBUF * q + b
          pltpu.make_async_copy(rows[b], acc_sh.at[dst_v.at[j]],
                                ssem[b]).wait()
          pltpu.async_copy(g_sh.at[src_v.at[j + NBUF]], rows[b], gsem[b])
        return carry

      lax.fori_loop(0, NQ - 1, step, 0)
      for b in range(NBUF):
        j = NBUF * (NQ - 1) + b
        pltpu.make_async_copy(g_sh.at[src_v.at[j]], rows[b], gsem[b]).wait()
        pltpu.async_copy(rows[b], acc_sh.at[dst_v.at[j]], ssem[b], add=True)
      for b in range(NBUF):
        j = NBUF * (NQ - 1) + b
        pltpu.make_async_copy(rows[b], acc_sh.at[dst_v.at[j]], ssem[b]).wait()

    plsc.subcore_barrier()
    pltpu.sync_copy(acc_sh.at[pl.ds(sid * RPT, RPT)],
                    out_hbm.at[cid, pl.ds(sid * RPT, RPT)])

  return scatter


_scatter_deg = _make_scatter(16, ones_mode=True)
_scatter16 = _make_scatter(16)
_scatter64 = _make_scatter(64)


def _k0_body(x_ref, degp_ref, w_ref, h_ref, g_ref, dis_ref, dinv_ref):
  deg = degp_ref[0, :, 0:1] + degp_ref[1, :, 0:1] + 1.0
  dis = lax.rsqrt(deg)
  dinv = 1.0 / deg
  h = jnp.dot(x_ref[...], w_ref[...], preferred_element_type=jnp.float32)
  h_ref[...] = h
  g_ref[...] = h * dis
  dis_ref[...] = dis
  dinv_ref[...] = dinv


def _mid_body(s_ref, h_ref, dis_ref, dinv_ref, b_ref, w_ref, hn_ref, gn_ref):
  dis = dis_ref[...]
  a = dis * (s_ref[0] + s_ref[1]) + dinv_ref[...] * h_ref[...] + b_ref[...]
  a = jnp.maximum(a, 0.0)
  h = jnp.dot(a, w_ref[...], preferred_element_type=jnp.float32)
  hn_ref[...] = h
  gn_ref[...] = h * dis


def _k3_body(s_ref, h_ref, dis_ref, dinv_ref, b_ref, out_ref):
  out_ref[...] = (dis_ref[...] * (s_ref[0] + s_ref[1])
                  + dinv_ref[...] * h_ref[...] + b_ref[...])


def _f32(shape):
  return jax.ShapeDtypeStruct(shape, jnp.float32)


_k0 = pl.pallas_call(
    _k0_body,
    out_shape=(_f32((R, 64)), _f32((R, 64)), _f32((R, 1)), _f32((R, 1))),
)


def _mid(dout):
  return pl.pallas_call(_mid_body, out_shape=(_f32((R, dout)), _f32((R, dout))))


_k1 = _mid(64)
_k2 = _mid(16)
_k3 = pl.pallas_call(_k3_body, out_shape=_f32((R, 16)))


def kernel(x, edge_index, W1, b1, W2, b2, W3, b3):
  ei = edge_index.astype(jnp.int32)
  ei = jnp.pad(ei, ((0, 0), (0, EPAD - E)), constant_values=N)
  ei = ei.reshape(2, NW, NCHUNK, CH)

  xp = jnp.zeros((R, 128), jnp.float32).at[:N].set(x)
  ones16 = jnp.ones((R, 16), jnp.float32)
  zero16 = jnp.zeros((RPT, 16), jnp.float32)
  zero64 = jnp.zeros((RPT, 64), jnp.float32)

  degp = _scatter_deg(ones16, ei, zero16)

  w3p = jnp.zeros((64, 16), jnp.float32).at[:, :6].set(W3)
  b1r = b1.reshape(1, 64)
  b2r = b2.reshape(1, 64)
  b3r = jnp.zeros((1, 16), jnp.float32).at[0, :6].set(b3)

  h1, g1, dis, dinv = _k0(xp, degp, W1)
  s1 = _scatter64(g1, ei, zero64)
  h2, g2 = _k1(s1, h1, dis, dinv, b1r, W2)
  s2 = _scatter64(g2, ei, zero64)
  h3, g3 = _k2(s2, h2, dis, dinv, b2r, w3p)
  s3 = _scatter16(g3, ei, zero16)
  out = _k3(s3, h3, dis, dinv, b3r)
  return out[:N, :6]
